# Initial kernel scaffold; baseline (speedup 1.0000x reference)
import jax
import jax.numpy as jnp
from jax.experimental import pallas as pl


def kernel(x_nchw, dw_w, pw_w, gamma, beta):
    raise NotImplementedError("write your pallas kernel here")


# speedup vs baseline: 1.7333x; 1.7333x over previous
import functools

import jax
import jax.numpy as jnp
from jax.experimental import pallas as pl
from jax.experimental.pallas import tpu as pltpu

_NB = 16
_PADL = 128


def _balanced_add(ts):
    n = len(ts)
    if n == 1:
        return ts[0]
    return _balanced_add(ts[: n // 2]) + _balanced_add(ts[n // 2:])


def _relu_dw_pw(x_ref, wp_ref, pmat_ref, pad_ref, *, hw, w):
    pad_ref[:, :_PADL] = jnp.zeros_like(pad_ref[:, :_PADL])
    pad_ref[:, _PADL + hw:] = jnp.zeros_like(pad_ref[:, _PADL + hw:])
    pad_ref[:, _PADL:_PADL + hw] = jnp.maximum(x_ref[...], 0.0)

    taps = []
    for kh in range(3):
        for kw in range(3):
            off = _PADL + (kh - 1) * w + (kw - 1)
            taps.append(pad_ref[:, off:off + hw] * wp_ref[kh * 3 + kw])
    acc = _balanced_add(taps)

    return jax.lax.dot_general(pmat_ref[...], acc,
                               (((1,), (0,)), ((), ())),
                               preferred_element_type=jnp.float32)


def _moments_kernel(x_ref, wp_ref, pmat_ref, mom_ref, pad_ref, *, hw, w):
    y = _relu_dw_pw(x_ref, wp_ref, pmat_ref, pad_ref, hw=hw, w=w)
    s = jnp.sum(y, axis=1, keepdims=True)
    q = jnp.sum(y * y, axis=1, keepdims=True)
    mom_ref[0] = jnp.concatenate([s, q], axis=1)


def _bn_apply_kernel(x_ref, wp_ref, pmat_ref, shift_ref, o_ref, pad_ref, *,
                     hw, w):
    y = _relu_dw_pw(x_ref, wp_ref, pmat_ref, pad_ref, hw=hw, w=w)
    o_ref[...] = y + shift_ref[...]


def kernel(x_nchw, dw_w, pw_w, gamma, beta):
    n, cin, h, w = x_nchw.shape
    cout = pw_w.shape[0]
    hw = h * w
    nb = _NB
    assert n % nb == 0
    grid = (n // nb,)
    lpad = _PADL + hw + 128
    rin = nb * cin
    rout = nb * cout
    eps = 1e-5

    x_flat = x_nchw.astype(jnp.float32).reshape(n * cin, hw)

    lane = jnp.arange(hw, dtype=jnp.int32) % w
    kwm = jnp.stack([(lane >= 1).astype(jnp.float32),
                     jnp.ones((hw,), jnp.float32),
                     (lane < w - 1).astype(jnp.float32)])
    tapw = dw_w.astype(jnp.float32).reshape(cin, 9).T
    masks = kwm[jnp.tile(jnp.arange(3), 3)]
    planes = tapw[:, :, None] * masks[:, None, :]
    planes = jnp.broadcast_to(planes[:, None], (9, nb, cin, hw))
    planes = planes.reshape(9, rin, hw)

    pmat = pw_w.astype(jnp.float32).reshape(cout, cin)
    eye_nb = jnp.eye(nb, dtype=jnp.float32)
    pbig = jnp.kron(eye_nb, pmat)

    cparams = pltpu.CompilerParams(dimension_semantics=("parallel",),
                                   vmem_limit_bytes=64 * 1024 * 1024)

    moments = pl.pallas_call(
        functools.partial(_moments_kernel, hw=hw, w=w),
        out_shape=jax.ShapeDtypeStruct((n // nb, rout, 2), jnp.float32),
        grid=grid,
        in_specs=[pl.BlockSpec((rin, hw), lambda i: (i, 0)),
                  pl.BlockSpec((9, rin, hw), lambda i: (0, 0, 0)),
                  pl.BlockSpec((rout, rin), lambda i: (0, 0))],
        out_specs=pl.BlockSpec((1, rout, 2), lambda i: (i, 0, 0)),
        scratch_shapes=[pltpu.VMEM((rin, lpad), jnp.float32)],
        compiler_params=cparams,
    )(x_flat, planes, pbig)

    tot = jnp.sum(moments.reshape(n // nb, nb, cout, 2), axis=(0, 1))
    count = jnp.float32(n * hw)
    mean = tot[:, 0:1] / count
    var = tot[:, 1:2] / count - mean * mean
    inv = jax.lax.rsqrt(var + eps)
    scale = gamma.astype(jnp.float32).reshape(cout, 1) * inv
    shift = beta.astype(jnp.float32).reshape(cout, 1) - mean * scale
    pbig_s = jnp.kron(eye_nb, pmat * scale)
    shift_big = jnp.tile(shift, (nb, 1))

    y = pl.pallas_call(
        functools.partial(_bn_apply_kernel, hw=hw, w=w),
        out_shape=jax.ShapeDtypeStruct((n * cout, hw), jnp.float32),
        grid=grid,
        in_specs=[pl.BlockSpec((rin, hw), lambda i: (i, 0)),
                  pl.BlockSpec((9, rin, hw), lambda i: (0, 0, 0)),
                  pl.BlockSpec((rout, rin), lambda i: (0, 0)),
                  pl.BlockSpec((rout, 1), lambda i: (0, 0))],
        out_specs=pl.BlockSpec((rout, hw), lambda i: (i, 0)),
        scratch_shapes=[pltpu.VMEM((rin, lpad), jnp.float32)],
        compiler_params=cparams,
    )(x_flat, planes, pbig_s, shift_big)

    return y.reshape(n, cout, h, w)


# speedup vs baseline: 14.2929x; 8.2459x over previous
import functools

import jax
import jax.numpy as jnp
from jax.experimental import pallas as pl
from jax.experimental.pallas import tpu as pltpu

_NL = 128


def _balanced_add(ts):
    n = len(ts)
    if n == 1:
        return ts[0]
    return _balanced_add(ts[: n // 2]) + _balanced_add(ts[n // 2:])


def _conv_ys(x_ref, dw_ref, pm_ref, xp_ref):
    cin, h, w, _ = x_ref.shape
    cout = pm_ref.shape[0]
    xp_ref[:, :, 0:1, :] = jnp.zeros_like(xp_ref[:, :, 0:1, :])
    xp_ref[:, :, w + 1:w + 2, :] = jnp.zeros_like(xp_ref[:, :, w + 1:w + 2, :])
    xp_ref[:, 0:1, :, :] = jnp.zeros_like(xp_ref[:, 0:1, :, :])
    xp_ref[:, h + 1:h + 2, :, :] = jnp.zeros_like(xp_ref[:, h + 1:h + 2, :, :])
    xp_ref[:, 1:h + 1, 1:w + 1, :] = jnp.maximum(x_ref[...], 0.0)

    ys = [None] * cout
    for ci in range(cin):
        taps = [xp_ref[ci, kh:kh + h, kw:kw + w, :] * dw_ref[ci, kh * 3 + kw]
                for kh in range(3) for kw in range(3)]
        acc = _balanced_add(taps)
        for co in range(cout):
            t = acc * pm_ref[co, ci]
            ys[co] = t if ci == 0 else ys[co] + t
    return ys


def _moments_kernel(x_ref, dw_ref, pm_ref, mom_ref, xp_ref):
    ys = _conv_ys(x_ref, dw_ref, pm_ref, xp_ref)
    for co, y in enumerate(ys):
        mom_ref[0, co, 0] = jnp.sum(y, axis=(0, 1))
        mom_ref[0, co, 1] = jnp.sum(y * y, axis=(0, 1))


def _bn_apply_kernel(x_ref, dw_ref, pm_ref, shift_ref, o_ref, xp_ref):
    ys = _conv_ys(x_ref, dw_ref, pm_ref, xp_ref)
    for co, y in enumerate(ys):
        o_ref[co] = y + shift_ref[co, 0]


def kernel(x_nchw, dw_w, pw_w, gamma, beta):
    n, cin, h, w = x_nchw.shape
    cout = pw_w.shape[0]
    nl = _NL
    assert n % nl == 0
    grid = (n // nl,)
    eps = 1e-5

    xt = jnp.transpose(x_nchw.astype(jnp.float32), (1, 2, 3, 0))

    dw = dw_w.astype(jnp.float32).reshape(cin, 9)
    pmat = pw_w.astype(jnp.float32).reshape(cout, cin)

    cparams = pltpu.CompilerParams(dimension_semantics=("parallel",),
                                   vmem_limit_bytes=64 * 1024 * 1024)
    smem = pl.BlockSpec(memory_space=pltpu.SMEM)

    moments = pl.pallas_call(
        _moments_kernel,
        out_shape=jax.ShapeDtypeStruct((n // nl, cout, 2, nl), jnp.float32),
        grid=grid,
        in_specs=[pl.BlockSpec((cin, h, w, nl), lambda i: (0, 0, 0, i)),
                  smem, smem],
        out_specs=pl.BlockSpec((1, cout, 2, nl), lambda i: (i, 0, 0, 0)),
        scratch_shapes=[pltpu.VMEM((cin, h + 2, w + 2, nl), jnp.float32)],
        compiler_params=cparams,
    )(xt, dw, pmat)

    tot = jnp.sum(moments, axis=(0, 3))
    count = jnp.float32(n * h * w)
    mean = tot[:, 0:1] / count
    var = tot[:, 1:2] / count - mean * mean
    inv = jax.lax.rsqrt(var + eps)
    scale = gamma.astype(jnp.float32).reshape(cout, 1) * inv
    shift = beta.astype(jnp.float32).reshape(cout, 1) - mean * scale
    pmat_s = pmat * scale

    yt = pl.pallas_call(
        _bn_apply_kernel,
        out_shape=jax.ShapeDtypeStruct((cout, h, w, n), jnp.float32),
        grid=grid,
        in_specs=[pl.BlockSpec((cin, h, w, nl), lambda i: (0, 0, 0, i)),
                  smem, smem, smem],
        out_specs=pl.BlockSpec((cout, h, w, nl), lambda i: (0, 0, 0, i)),
        scratch_shapes=[pltpu.VMEM((cin, h + 2, w + 2, nl), jnp.float32)],
        compiler_params=cparams,
    )(xt, dw, pmat_s, shift)

    return jnp.transpose(yt, (3, 0, 1, 2))
